# baseline (device time: 23371 ns/iter reference)
import os

import jax
import jax.numpy as jnp
from jax import lax
from jax.experimental import pallas as pl
from jax.experimental.pallas import tpu as pltpu

N_DEV = 8
CH = int(os.environ.get("KERNEL_CH", "512"))
ROWSUM = os.environ.get("KERNEL_ROWSUM", "vpu")
COMM = os.environ.get("KERNEL_COMM", "2wave")
DB_MXU = os.environ.get("KERNEL_DB") == "mxu"
NO_COMM = os.environ.get("KERNEL_NO_COMM") == "1"


def kernel(x, dy, gamma):
    m, d = x.shape
    nsteps = m // CH
    assert nsteps >= 2

    def body(x_ref, dy_ref, gamma_ref, out_ref, acc_ref, sbuf_ref,
             gather_ref, send_sems, recv_sems):
        c = pl.program_id(0)
        my = lax.axis_index("i")
        barrier_sem = None if NO_COMM else pltpu.get_barrier_semaphore()

        def make_rdma(wave, j):
            return pltpu.make_async_remote_copy(
                src_ref=sbuf_ref.at[wave],
                dst_ref=gather_ref.at[wave, j - 1],
                send_sem=send_sems.at[wave, j - 1],
                recv_sem=recv_sems.at[wave, j - 1],
                device_id=(lax.rem(my + j, N_DEV),),
                device_id_type=pl.DeviceIdType.MESH,
            )

        @pl.when(c == 0)
        def _():
            acc_ref[:, :] = jnp.zeros((2, d), jnp.float32)
            if not NO_COMM:
                for j in range(1, N_DEV):
                    pl.semaphore_signal(
                        barrier_sem, inc=1,
                        device_id=(lax.rem(my + j, N_DEV),),
                        device_id_type=pl.DeviceIdType.MESH,
                    )

        xs = x_ref[:, :]
        dys = dy_ref[:, :]
        if ROWSUM == "mxu":
            ones_col = jnp.ones((d, 1), jnp.float32)
            x_sum = jnp.dot(xs, ones_col, preferred_element_type=jnp.float32)
            x2_sum = jnp.dot(
                xs * xs, ones_col, preferred_element_type=jnp.float32)
        else:
            x_sum = jnp.sum(xs, axis=1, keepdims=True)
            x2_sum = jnp.sum(xs * xs, axis=1, keepdims=True)
        mu = x_sum * (1.0 / d)
        var = x2_sum * (1.0 / d) - mu * mu
        rstd = lax.rsqrt(var + 1e-5)
        b = mu * rstd
        dg = (jnp.sum(xs * (dys * rstd), axis=0, keepdims=True)
              - jnp.sum(dys * b, axis=0, keepdims=True))
        if DB_MXU:
            db = jnp.dot(jnp.ones((1, CH), jnp.float32), dys,
                         preferred_element_type=jnp.float32)
        else:
            db = jnp.sum(dys, axis=0, keepdims=True)
        acc_ref[0:1, :] += dg
        acc_ref[1:2, :] += db

        if NO_COMM:
            @pl.when(c == nsteps - 1)
            def _():
                out_ref[:, :] = acc_ref[:, :]
            return

        if COMM == "2wave":
            @pl.when(c == 1)
            def _():
                pl.semaphore_wait(barrier_sem, N_DEV - 1)
                sbuf_ref[0] = acc_ref[:, :]
                for j in range(1, N_DEV):
                    make_rdma(0, j).start()

            @pl.when(c == nsteps - 1)
            def _():
                sbuf_ref[1] = acc_ref[:, :] - sbuf_ref[0]
                for j in range(1, N_DEV):
                    make_rdma(1, j).start()
                total = acc_ref[:, :]
                for w in (0, 1):
                    for j in range(1, N_DEV):
                        make_rdma(w, j).wait_recv()
                        total = total + gather_ref[w, j - 1, :, :]
                for w in (0, 1):
                    for j in range(1, N_DEV):
                        make_rdma(w, j).wait_send()
                out_ref[:, :] = total
        else:
            @pl.when(c == nsteps - 1)
            def _():
                pl.semaphore_wait(barrier_sem, N_DEV - 1)
                sbuf_ref[0] = acc_ref[:, :]
                for j in range(1, N_DEV):
                    make_rdma(0, j).start()
                total = acc_ref[:, :]
                for j in range(1, N_DEV):
                    make_rdma(0, j).wait_recv()
                    total = total + gather_ref[0, j - 1, :, :]
                for j in range(1, N_DEV):
                    make_rdma(0, j).wait_send()
                out_ref[:, :] = total

    return pl.pallas_call(
        body,
        grid=(nsteps,),
        out_shape=jax.ShapeDtypeStruct((2, d), jnp.float32),
        in_specs=[
            pl.BlockSpec((CH, d), lambda c: (c, 0)),
            pl.BlockSpec((CH, d), lambda c: (c, 0)),
            pl.BlockSpec((d,), lambda c: (0,)),
        ],
        out_specs=pl.BlockSpec((2, d), lambda c: (0, 0)),
        scratch_shapes=[
            pltpu.VMEM((2, d), jnp.float32),
            pltpu.VMEM((2, 2, d), jnp.float32),
            pltpu.VMEM((2, N_DEV - 1, 2, d), jnp.float32),
            pltpu.SemaphoreType.DMA((2, N_DEV - 1)),
            pltpu.SemaphoreType.DMA((2, N_DEV - 1)),
        ],
        compiler_params=pltpu.CompilerParams(
            collective_id=None if NO_COMM else 0,
            dimension_semantics=("arbitrary",),
        ),
    )(x, dy, gamma)


# device time: 22116 ns/iter; 1.0567x vs baseline; 1.0567x over previous
import os

import jax
import jax.numpy as jnp
from jax import lax
from jax.experimental import pallas as pl
from jax.experimental.pallas import tpu as pltpu

N_DEV = 8
CH = int(os.environ.get("KERNEL_CH", "512"))
ROWSUM = os.environ.get("KERNEL_ROWSUM", "vpu")
COMM = os.environ.get("KERNEL_COMM", "2wave")
DB_MXU = os.environ.get("KERNEL_DB") == "mxu"
NO_COMM = os.environ.get("KERNEL_NO_COMM") == "1"


def kernel(x, dy, gamma):
    m, d = x.shape
    nsteps = m // CH
    assert nsteps >= 2

    def body(x_ref, dy_ref, gamma_ref, out_ref, acc_ref, sbuf_ref,
             gather_ref, send_sems, recv_sems):
        c = pl.program_id(0)
        my = lax.axis_index("i")
        barrier_sem = None if NO_COMM else pltpu.get_barrier_semaphore()

        def make_rdma(wave, j):
            return pltpu.make_async_remote_copy(
                src_ref=sbuf_ref.at[wave],
                dst_ref=gather_ref.at[wave, j - 1],
                send_sem=send_sems.at[wave, j - 1],
                recv_sem=recv_sems.at[wave, j - 1],
                device_id=(lax.rem(my + j, N_DEV),),
                device_id_type=pl.DeviceIdType.MESH,
            )

        @pl.when(c == 0)
        def _():
            acc_ref[:, :] = jnp.zeros((2, d), jnp.float32)
            if not NO_COMM:
                for j in range(1, N_DEV):
                    pl.semaphore_signal(
                        barrier_sem, inc=1,
                        device_id=(lax.rem(my + j, N_DEV),),
                        device_id_type=pl.DeviceIdType.MESH,
                    )

        xs = x_ref[:, :]
        dys = dy_ref[:, :]
        if ROWSUM == "mxu":
            ones_col = jnp.ones((d, 1), jnp.float32)
            x_sum = jnp.dot(xs, ones_col, preferred_element_type=jnp.float32)
            x2_sum = jnp.dot(
                xs * xs, ones_col, preferred_element_type=jnp.float32)
        else:
            x_sum = jnp.sum(xs, axis=1, keepdims=True)
            x2_sum = jnp.sum(xs * xs, axis=1, keepdims=True)
        mu = x_sum * (1.0 / d)
        var = x2_sum * (1.0 / d) - mu * mu
        rstd = lax.rsqrt(var + 1e-5)
        b = mu * rstd
        dg = jnp.sum(dys * (xs * rstd - b), axis=0, keepdims=True)
        if DB_MXU:
            db = jnp.dot(jnp.ones((1, CH), jnp.float32), dys,
                         preferred_element_type=jnp.float32)
        else:
            db = jnp.sum(dys, axis=0, keepdims=True)
        acc_ref[0:1, :] += dg
        acc_ref[1:2, :] += db

        if NO_COMM:
            @pl.when(c == nsteps - 1)
            def _():
                out_ref[:, :] = acc_ref[:, :]
            return

        if COMM == "2wave":
            @pl.when(c == nsteps - 2)
            def _():
                pl.semaphore_wait(barrier_sem, N_DEV - 1)
                sbuf_ref[0] = acc_ref[:, :]
                for j in range(1, N_DEV):
                    make_rdma(0, j).start()

            @pl.when(c == nsteps - 1)
            def _():
                sbuf_ref[1] = acc_ref[:, :] - sbuf_ref[0]
                for j in range(1, N_DEV):
                    make_rdma(1, j).start()
                total = acc_ref[:, :]
                for w in (0, 1):
                    for j in range(1, N_DEV):
                        make_rdma(w, j).wait_recv()
                        total = total + gather_ref[w, j - 1, :, :]
                for w in (0, 1):
                    for j in range(1, N_DEV):
                        make_rdma(w, j).wait_send()
                out_ref[:, :] = total
        else:
            @pl.when(c == nsteps - 1)
            def _():
                pl.semaphore_wait(barrier_sem, N_DEV - 1)
                sbuf_ref[0] = acc_ref[:, :]
                for j in range(1, N_DEV):
                    make_rdma(0, j).start()
                total = acc_ref[:, :]
                for j in range(1, N_DEV):
                    make_rdma(0, j).wait_recv()
                    total = total + gather_ref[0, j - 1, :, :]
                for j in range(1, N_DEV):
                    make_rdma(0, j).wait_send()
                out_ref[:, :] = total

    return pl.pallas_call(
        body,
        grid=(nsteps,),
        out_shape=jax.ShapeDtypeStruct((2, d), jnp.float32),
        in_specs=[
            pl.BlockSpec((CH, d), lambda c: (c, 0)),
            pl.BlockSpec((CH, d), lambda c: (c, 0)),
            pl.BlockSpec((d,), lambda c: (0,)),
        ],
        out_specs=pl.BlockSpec((2, d), lambda c: (0, 0)),
        scratch_shapes=[
            pltpu.VMEM((2, d), jnp.float32),
            pltpu.VMEM((2, 2, d), jnp.float32),
            pltpu.VMEM((2, N_DEV - 1, 2, d), jnp.float32),
            pltpu.SemaphoreType.DMA((2, N_DEV - 1)),
            pltpu.SemaphoreType.DMA((2, N_DEV - 1)),
        ],
        compiler_params=pltpu.CompilerParams(
            collective_id=None if NO_COMM else 0,
            dimension_semantics=("arbitrary",),
        ),
    )(x, dy, gamma)


# device time: 19397 ns/iter; 1.2049x vs baseline; 1.1402x over previous
import os

import jax
import jax.numpy as jnp
from jax import lax
from jax.experimental import pallas as pl
from jax.experimental.pallas import tpu as pltpu

N_DEV = 8
CH = int(os.environ.get("KERNEL_CH", "512"))
ROWSUM = os.environ.get("KERNEL_ROWSUM", "vpu")
COMM = os.environ.get("KERNEL_COMM", "2wave")
DB_MXU = os.environ.get("KERNEL_DB") == "mxu"
NO_COMM = os.environ.get("KERNEL_NO_COMM") == "1"
PAD_MB = int(os.environ.get("KERNEL_PAD_MB", "0"))


def kernel(x, dy, gamma):
    m, d = x.shape
    nsteps = m // CH
    assert nsteps >= 2

    def body(x_ref, dy_ref, gamma_ref, out_ref, acc_ref, sbuf_ref,
             gather_ref, send_sems, recv_sems, *pad):
        c = pl.program_id(0)
        my = lax.axis_index("i")
        barrier_sem = None if NO_COMM else pltpu.get_barrier_semaphore()

        def make_rdma(wave, j):
            return pltpu.make_async_remote_copy(
                src_ref=sbuf_ref.at[wave],
                dst_ref=gather_ref.at[wave, j - 1],
                send_sem=send_sems.at[wave, j - 1],
                recv_sem=recv_sems.at[wave, j - 1],
                device_id=(lax.rem(my + j, N_DEV),),
                device_id_type=pl.DeviceIdType.MESH,
            )

        @pl.when(c == 0)
        def _():
            acc_ref[:, :] = jnp.zeros((2, d), jnp.float32)
            if not NO_COMM:
                for j in range(1, N_DEV):
                    pl.semaphore_signal(
                        barrier_sem, inc=1,
                        device_id=(lax.rem(my + j, N_DEV),),
                        device_id_type=pl.DeviceIdType.MESH,
                    )

        xs = x_ref[:, :]
        dys = dy_ref[:, :]
        if ROWSUM == "mxu":
            ones_col = jnp.ones((d, 1), jnp.float32)
            x_sum = jnp.dot(xs, ones_col, preferred_element_type=jnp.float32)
            x2_sum = jnp.dot(
                xs * xs, ones_col, preferred_element_type=jnp.float32)
        else:
            x_sum = jnp.sum(xs, axis=1, keepdims=True)
            x2_sum = jnp.sum(xs * xs, axis=1, keepdims=True)
        mu = x_sum * (1.0 / d)
        var = x2_sum * (1.0 / d) - mu * mu
        rstd = lax.rsqrt(var + 1e-5)
        b = mu * rstd
        dg = jnp.sum(dys * (xs * rstd - b), axis=0, keepdims=True)
        if DB_MXU:
            db = jnp.dot(jnp.ones((1, CH), jnp.float32), dys,
                         preferred_element_type=jnp.float32)
        else:
            db = jnp.sum(dys, axis=0, keepdims=True)
        acc_ref[0:1, :] += dg
        acc_ref[1:2, :] += db

        if NO_COMM:
            @pl.when(c == nsteps - 1)
            def _():
                out_ref[:, :] = acc_ref[:, :]
            return

        if COMM == "2wave":
            @pl.when(c == nsteps - 2)
            def _():
                pl.semaphore_wait(barrier_sem, N_DEV - 1)
                sbuf_ref[0] = acc_ref[:, :]
                for j in range(1, N_DEV):
                    make_rdma(0, j).start()

            @pl.when(c == nsteps - 1)
            def _():
                sbuf_ref[1] = acc_ref[:, :] - sbuf_ref[0]
                for j in range(1, N_DEV):
                    make_rdma(1, j).start()
                total = acc_ref[:, :]
                for w in (0, 1):
                    for j in range(1, N_DEV):
                        make_rdma(w, j).wait_recv()
                        total = total + gather_ref[w, j - 1, :, :]
                for w in (0, 1):
                    for j in range(1, N_DEV):
                        make_rdma(w, j).wait_send()
                out_ref[:, :] = total
        else:
            @pl.when(c == nsteps - 1)
            def _():
                pl.semaphore_wait(barrier_sem, N_DEV - 1)
                sbuf_ref[0] = acc_ref[:, :]
                for j in range(1, N_DEV):
                    make_rdma(0, j).start()
                total = acc_ref[:, :]
                for j in range(1, N_DEV):
                    make_rdma(0, j).wait_recv()
                    total = total + gather_ref[0, j - 1, :, :]
                for j in range(1, N_DEV):
                    make_rdma(0, j).wait_send()
                out_ref[:, :] = total

    if os.environ.get("KERNEL_PIN", "hbm") == "hbm":
        x = pltpu.with_memory_space_constraint(x, pltpu.MemorySpace.HBM)
        dy = pltpu.with_memory_space_constraint(dy, pltpu.MemorySpace.HBM)
        gamma = pltpu.with_memory_space_constraint(
            gamma, pltpu.MemorySpace.HBM)

    return pl.pallas_call(
        body,
        grid=(nsteps,),
        out_shape=jax.ShapeDtypeStruct((2, d), jnp.float32),
        in_specs=[
            pl.BlockSpec((CH, d), lambda c: (c, 0)),
            pl.BlockSpec((CH, d), lambda c: (c, 0)),
            pl.BlockSpec((d,), lambda c: (0,)),
        ],
        out_specs=pl.BlockSpec((2, d), lambda c: (0, 0)),
        scratch_shapes=[
            pltpu.VMEM((2, d), jnp.float32),
            pltpu.VMEM((2, 2, d), jnp.float32),
            pltpu.VMEM((2, N_DEV - 1, 2, d), jnp.float32),
            pltpu.SemaphoreType.DMA((2, N_DEV - 1)),
            pltpu.SemaphoreType.DMA((2, N_DEV - 1)),
        ] + ([pltpu.VMEM((PAD_MB * 2048, 128), jnp.float32)]
             if PAD_MB else []),
        compiler_params=pltpu.CompilerParams(
            collective_id=None if NO_COMM else 0,
            dimension_semantics=("arbitrary",),
        ),
    )(x, dy, gamma)
